# Initial kernel scaffold; baseline (speedup 1.0000x reference)
import jax
import jax.numpy as jnp
from jax.experimental import pallas as pl


def kernel(input_ids, tables):
    raise NotImplementedError("write your pallas kernel here")


# speedup vs baseline: 1.8892x; 1.8892x over previous
import functools

import jax
import jax.numpy as jnp
from jax import lax
from jax.experimental import pallas as pl
from jax.experimental.pallas import tpu as pltpu
from jax.experimental.pallas import tpu_sc as plsc


def _make_sc_kernel(R, H, L, C):
    info = plsc.get_sparse_core_info()
    NC, NS, LANES = info.num_cores, info.num_subcores, info.num_lanes
    NW = NC * NS
    assert R % (NW * C) == 0
    rows_per_w = R // NW
    n_chunks = rows_per_w // C
    mesh = plsc.VectorSubcoreMesh(core_axis_name="c", subcore_axis_name="s")

    @functools.partial(
        pl.kernel,
        mesh=mesh,
        out_type=jax.ShapeDtypeStruct((R, H), jnp.float32),
        scratch_types=[
            pltpu.VMEM((C * L,), jnp.int32),
            pltpu.VMEM((C * L, H), jnp.float32),
            pltpu.VMEM((C, H), jnp.float32),
            pltpu.SemaphoreType.DMA,
        ],
    )
    def k(idx_hbm, tables_hbm, out_hbm, idx_v, rows_v, acc_v, sem):
        wid = lax.axis_index("s") * NC + lax.axis_index("c")
        base = wid * rows_per_w

        def chunk_body(ci, carry):
            rowbase = base + ci * C
            pltpu.sync_copy(idx_hbm.at[pl.ds(rowbase * L, C * L)], idx_v)
            pltpu.async_copy(tables_hbm.at[idx_v], rows_v, sem).wait()

            def h_body(hi, carry2):
                off = pl.multiple_of(hi * LANES, LANES)
                for c in range(C):
                    acc = rows_v[c * L, pl.ds(off, LANES)]
                    for l in range(1, L):
                        acc = acc + rows_v[c * L + l, pl.ds(off, LANES)]
                    acc_v[c, pl.ds(off, LANES)] = acc
                return carry2

            lax.fori_loop(0, H // LANES, h_body, 0)
            pltpu.sync_copy(acc_v, out_hbm.at[pl.ds(rowbase, C)])
            return carry

        lax.fori_loop(0, n_chunks, chunk_body, 0)

    return k


def kernel(input_ids, tables):
    B, L, S = input_ids.shape
    _, V, H = tables.shape
    R = B * S
    ids32 = input_ids.astype(jnp.int32)
    flat_idx = ids32 + (jnp.arange(L, dtype=jnp.int32) * V)[None, :, None]
    flat_idx = flat_idx.transpose(0, 2, 1).reshape(R * L)
    tab = tables.reshape(L * V, H)
    out = _make_sc_kernel(R, H, L, C=8)(flat_idx, tab)
    return out.reshape(B, S, H)
